# baseline (speedup 1.0000x reference)
import functools

import jax
import jax.numpy as jnp
from jax import lax
from jax.experimental import pallas as pl
from jax.experimental.pallas import tpu as pltpu
from jax.experimental.pallas import tpu_sc as plsc

N_USERS = 1000000
D = 32
B = 16384
NC = 2
NS = 16
NW = NC * NS
BPW = B // NW
GCHUNK = 128
NGC = BPW // GCHUNK
L = 16


@jax.jit
def _gmf_sc(users, items, embed_t, wb):
    uidx = users.reshape(NW, NGC, GCHUNK).astype(jnp.int32)
    iidx = items.reshape(NW, NGC, GCHUNK).astype(jnp.int32)

    mesh = plsc.VectorSubcoreMesh(core_axis_name="c", subcore_axis_name="s")

    @functools.partial(
        pl.kernel,
        mesh=mesh,
        out_type=jax.ShapeDtypeStruct((B,), jnp.float32),
        compiler_params=pltpu.CompilerParams(use_tc_tiling_on_sc=False),
        scratch_types=[
            pltpu.VMEM((NGC, GCHUNK), jnp.int32),
            pltpu.VMEM((NGC, GCHUNK), jnp.int32),
            pltpu.VMEM((D, BPW), jnp.float32),
            pltpu.VMEM((D, BPW), jnp.float32),
            pltpu.VMEM((D, L), jnp.float32),
            pltpu.VMEM((BPW,), jnp.float32),
            pltpu.SemaphoreType.DMA,
        ],
    )
    def run(users_hbm, items_hbm, table_hbm, wb_hbm, out_hbm,
            uidx_v, iidx_v, ucols_v, icols_v, wb_v, out_v, sem):
        wid = lax.axis_index("s") * NC + lax.axis_index("c")
        pltpu.sync_copy(users_hbm.at[wid], uidx_v)
        pltpu.sync_copy(items_hbm.at[wid], iidx_v)
        pltpu.sync_copy(wb_hbm, wb_v)

        copies = []
        for d in range(D):
            for g in range(NGC):
                dst = pl.ds(g * GCHUNK, GCHUNK)
                copies.append(pltpu.async_copy(
                    table_hbm.at[d].at[uidx_v.at[g]], ucols_v.at[d, dst], sem))
                copies.append(pltpu.async_copy(
                    table_hbm.at[d].at[iidx_v.at[g]], icols_v.at[d, dst], sem))
        for c in copies:
            c.wait()

        def group_step(m, _):
            col = pl.ds(m * L, L)
            acc = jnp.zeros((L,), jnp.float32)
            for d in range(D):
                acc = acc + ucols_v[d, col] * icols_v[d, col] * wb_v[d, :]
            out_v[col] = acc
            return 0

        lax.fori_loop(0, BPW // L, group_step, 0)

        pltpu.sync_copy(out_v, out_hbm.at[pl.ds(wid * BPW, BPW)])

    return run(uidx, iidx, embed_t, wb)


def kernel(users, items, embed_user, W):
    wb = jnp.broadcast_to(W.reshape(D, 1), (D, L))
    return _gmf_sc(users, items, embed_user.T, wb)


# speedup vs baseline: 5.0845x; 5.0845x over previous
import functools

import jax
import jax.numpy as jnp
from jax import lax
from jax.experimental import pallas as pl
from jax.experimental.pallas import tpu as pltpu
from jax.experimental.pallas import tpu_sc as plsc

N_USERS = 1000000
D = 32
B = 16384
NC = 2
NS = 16
NW = NC * NS
BPW = B // NW
GCHUNK = 128
NGC = BPW // GCHUNK
L = 16


def _perm(x, idx):
    dnums = lax.GatherDimensionNumbers(
        offset_dims=(), collapsed_slice_dims=(0,), start_index_map=(0,))
    return lax.gather(x, idx[:, None], dnums, slice_sizes=(1,),
                      mode=lax.GatherScatterMode.PROMISE_IN_BOUNDS)


@jax.jit
def _gmf_sc(users, items, embed_user, W):
    mesh = plsc.VectorSubcoreMesh(core_axis_name="c", subcore_axis_name="s")

    @functools.partial(
        pl.kernel,
        mesh=mesh,
        out_type=jax.ShapeDtypeStruct((B,), jnp.float32),
        compiler_params=pltpu.CompilerParams(use_tc_tiling_on_sc=False),
        scratch_types=[
            pltpu.VMEM((BPW,), jnp.int32),
            pltpu.VMEM((BPW,), jnp.int32),
            pltpu.VMEM((BPW, D), jnp.float32),
            pltpu.VMEM((BPW, D), jnp.float32),
            pltpu.VMEM((1, D), jnp.float32),
            pltpu.VMEM((BPW,), jnp.float32),
            pltpu.SemaphoreType.DMA,
        ],
    )
    def run(users_hbm, items_hbm, table_hbm, w_hbm, out_hbm,
            uidx_v, iidx_v, urows_v, irows_v, w_v, out_v, sem):
        wid = lax.axis_index("s") * NC + lax.axis_index("c")
        base = wid * BPW
        pltpu.sync_copy(users_hbm.at[pl.ds(base, BPW)], uidx_v)
        pltpu.sync_copy(items_hbm.at[pl.ds(base, BPW)], iidx_v)
        pltpu.sync_copy(w_hbm, w_v)

        copies = []
        for g in range(NGC):
            rows = pl.ds(g * GCHUNK, GCHUNK)
            copies.append(pltpu.async_copy(
                table_hbm.at[uidx_v.at[rows]], urows_v.at[rows], sem))
            copies.append(pltpu.async_copy(
                table_hbm.at[iidx_v.at[rows]], irows_v.at[rows], sem))
        for c in copies:
            c.wait()

        w0 = w_v[0, pl.ds(0, L)]
        w1 = w_v[0, pl.ds(L, L)]

        lanes = lax.iota(jnp.int32, L)
        rot8 = lanes ^ 8
        rot4 = lanes ^ 4
        rot2 = lanes ^ 2
        rot1 = lanes ^ 1

        def chunk_step(c, _):
            r0 = c * L
            acc = jnp.zeros((L,), jnp.float32)
            for l in range(L):
                r = r0 + l
                u0 = urows_v[r, pl.ds(0, L)]
                u1 = urows_v[r, pl.ds(L, L)]
                i0 = irows_v[r, pl.ds(0, L)]
                i1 = irows_v[r, pl.ds(L, L)]
                s = u0 * i0 * w0 + u1 * i1 * w1
                s = s + _perm(s, rot8)
                s = s + _perm(s, rot4)
                s = s + _perm(s, rot2)
                s = s + _perm(s, rot1)
                acc = jnp.where(lanes == l, s, acc)
            out_v[pl.ds(r0, L)] = acc
            return 0

        lax.fori_loop(0, BPW // L, chunk_step, 0)

        pltpu.sync_copy(out_v, out_hbm.at[pl.ds(base, BPW)])

    return run(users.astype(jnp.int32), items.astype(jnp.int32),
               embed_user, W)


def kernel(users, items, embed_user, W):
    return _gmf_sc(users, items, embed_user, W)


# speedup vs baseline: 7.2310x; 1.4222x over previous
import functools

import jax
import jax.numpy as jnp
from jax import lax
from jax.experimental import pallas as pl
from jax.experimental.pallas import tpu as pltpu
from jax.experimental.pallas import tpu_sc as plsc

N_USERS = 1000000
D = 32
B = 16384
NC = 2
NS = 16
NW = NC * NS
BPW = B // NW
L = 16
NWAVE = BPW // L


def _perm(x, idx):
    dnums = lax.GatherDimensionNumbers(
        offset_dims=(), collapsed_slice_dims=(0,), start_index_map=(0,))
    return lax.gather(x, idx[:, None], dnums, slice_sizes=(1,),
                      mode=lax.GatherScatterMode.PROMISE_IN_BOUNDS)


@jax.jit
def _gmf_sc(users, items, embed_user, W):
    mesh = plsc.VectorSubcoreMesh(core_axis_name="c", subcore_axis_name="s")

    @functools.partial(
        pl.kernel,
        mesh=mesh,
        out_type=jax.ShapeDtypeStruct((B,), jnp.float32),
        compiler_params=pltpu.CompilerParams(use_tc_tiling_on_sc=True),
        scratch_types=[
            pltpu.VMEM((BPW,), jnp.int32),
            pltpu.VMEM((BPW,), jnp.int32),
            pltpu.VMEM((2, L, 8, D), jnp.float32),
            pltpu.VMEM((2, L, 8, D), jnp.float32),
            pltpu.VMEM((1, D), jnp.float32),
            pltpu.VMEM((BPW,), jnp.float32),
            pltpu.SemaphoreType.DMA,
            pltpu.SemaphoreType.DMA,
        ],
    )
    def run(users_hbm, items_hbm, table_hbm, w_hbm, out_hbm,
            uidx_v, iidx_v, ublk_v, iblk_v, w_v, out_v, sem0, sem1):
        wid = lax.axis_index("s") * NC + lax.axis_index("c")
        base = wid * BPW
        pltpu.sync_copy(users_hbm.at[pl.ds(base, BPW)], uidx_v)
        pltpu.sync_copy(items_hbm.at[pl.ds(base, BPW)], iidx_v)
        pltpu.sync_copy(w_hbm, w_v)

        def enqueue_wave(w, p, sem):
            uvec = uidx_v[pl.ds(w * L, L)]
            ivec = iidx_v[pl.ds(w * L, L)]
            for l in range(L):
                vu = uvec[l]
                offu = pl.multiple_of((vu >> 3) << 3, 8)
                pltpu.async_copy(
                    table_hbm.at[pl.ds(offu, 8), :], ublk_v.at[p, l], sem)
                vi = ivec[l]
                offi = pl.multiple_of((vi >> 3) << 3, 8)
                pltpu.async_copy(
                    table_hbm.at[pl.ds(offi, 8), :], iblk_v.at[p, l], sem)

        def drain_wave(p, sem):
            for l in range(L):
                pltpu.make_async_copy(
                    table_hbm.at[pl.ds(0, 8), :], ublk_v.at[p, l], sem).wait()
                pltpu.make_async_copy(
                    table_hbm.at[pl.ds(0, 8), :], iblk_v.at[p, l], sem).wait()

        w0 = w_v[0, pl.ds(0, L)]
        w1 = w_v[0, pl.ds(L, L)]
        lanes = lax.iota(jnp.int32, L)
        rot8 = lanes ^ 8
        rot4 = lanes ^ 4
        rot2 = lanes ^ 2
        rot1 = lanes ^ 1

        enqueue_wave(0, 0, sem0)

        def pair_step(h, _):
            we = h * 2
            wo = we + 1

            enqueue_wave(wo, 1, sem1)
            drain_wave(0, sem0)
            compute_wave(we, 0)

            @pl.when(h < NWAVE // 2 - 1)
            def _next_even():
                enqueue_wave(wo + 1, 0, sem0)

            drain_wave(1, sem1)
            compute_wave(wo, 1)
            return 0

        def compute_wave(w, p):
            uvec = uidx_v[pl.ds(w * L, L)]
            ivec = iidx_v[pl.ds(w * L, L)]
            acc = jnp.zeros((L,), jnp.float32)
            for l in range(L):
                ru = uvec[l] & 7
                ri = ivec[l] & 7
                u0 = ublk_v[p, l, ru, pl.ds(0, L)]
                u1 = ublk_v[p, l, ru, pl.ds(L, L)]
                i0 = iblk_v[p, l, ri, pl.ds(0, L)]
                i1 = iblk_v[p, l, ri, pl.ds(L, L)]
                s = u0 * i0 * w0 + u1 * i1 * w1
                s = s + _perm(s, rot8)
                s = s + _perm(s, rot4)
                s = s + _perm(s, rot2)
                s = s + _perm(s, rot1)
                acc = jnp.where(lanes == l, s, acc)
            out_v[pl.ds(w * L, L)] = acc

        lax.fori_loop(0, NWAVE // 2, pair_step, 0)

        pltpu.sync_copy(out_v, out_hbm.at[pl.ds(base, BPW)])

    return run(users.astype(jnp.int32), items.astype(jnp.int32),
               embed_user, W)


def kernel(users, items, embed_user, W):
    return _gmf_sc(users, items, embed_user, W)
